# baseline (speedup 1.0000x reference)
import jax
import jax.numpy as jnp
from jax.experimental import pallas as pl

EMBED = 4096
NUM_EXPERTS = 64
TOPK = 8
BLK = 512


def _gating_kernel(q_ref, w_ref, gates_ref, idx_ref):
    q = q_ref[...]
    w = w_ref[...]
    lt = jax.lax.dot_general(w, q, (((0,), (1,)), ((), ())),
                             preferred_element_type=jnp.float32)
    m = jnp.max(lt, axis=0, keepdims=True)
    e = jnp.exp(lt - m)
    p = e / jnp.sum(e, axis=0, keepdims=True)

    iota = jax.lax.broadcasted_iota(jnp.int32, p.shape, 0)
    vals = p
    grows = []
    irows = []
    for _ in range(TOPK):
        mx = jnp.max(vals, axis=0, keepdims=True)
        amx = jnp.min(jnp.where(vals == mx, iota, NUM_EXPERTS), axis=0,
                      keepdims=True)
        grows.append(mx)
        irows.append(amx)
        vals = jnp.where(iota == amx, -jnp.inf, vals)
    gt = jnp.concatenate(grows, axis=0)
    it = jnp.concatenate(irows, axis=0)
    gates_ref[...] = gt.T
    idx_ref[...] = it.T


def kernel(query, w_gate):
    B, A, P, D = query.shape
    tokens = B * A * P
    query_flat = query.reshape(tokens, D)
    grid = (tokens // BLK,)
    gates, idx = pl.pallas_call(
        _gating_kernel,
        grid=grid,
        in_specs=[
            pl.BlockSpec((BLK, EMBED), lambda i: (i, 0)),
            pl.BlockSpec((EMBED, NUM_EXPERTS), lambda i: (0, 0)),
        ],
        out_specs=[
            pl.BlockSpec((BLK, TOPK), lambda i: (i, 0)),
            pl.BlockSpec((BLK, TOPK), lambda i: (i, 0)),
        ],
        out_shape=[
            jax.ShapeDtypeStruct((tokens, TOPK), jnp.float32),
            jax.ShapeDtypeStruct((tokens, TOPK), jnp.int32),
        ],
    )(query_flat, w_gate)
    return (gates, idx)


# speedup vs baseline: 1.0697x; 1.0697x over previous
import jax
import jax.numpy as jnp
from jax.experimental import pallas as pl

EMBED = 4096
NUM_EXPERTS = 64
TOPK = 8
BLK = 1024


def _gating_kernel(q_ref, w_ref, gates_ref, idx_ref):
    q = q_ref[...]
    w = w_ref[...]
    lt = jax.lax.dot_general(w, q, (((0,), (1,)), ((), ())),
                             preferred_element_type=jnp.float32)
    m = jnp.max(lt, axis=0, keepdims=True)
    e = jnp.exp(lt - m)
    p = e / jnp.sum(e, axis=0, keepdims=True)

    iota = jax.lax.broadcasted_iota(jnp.int32, p.shape, 0)
    vals = p
    grows = []
    irows = []
    for _ in range(TOPK):
        mx = jnp.max(vals, axis=0, keepdims=True)
        amx = jnp.min(jnp.where(vals == mx, iota, NUM_EXPERTS), axis=0,
                      keepdims=True)
        grows.append(mx)
        irows.append(amx)
        vals = jnp.where(iota == amx, -jnp.inf, vals)
    gt = jnp.concatenate(grows, axis=0)
    it = jnp.concatenate(irows, axis=0)
    gates_ref[...] = gt.T
    idx_ref[...] = it.T


def kernel(query, w_gate):
    B, A, P, D = query.shape
    tokens = B * A * P
    query_flat = query.reshape(tokens, D)
    grid = (tokens // BLK,)
    gates, idx = pl.pallas_call(
        _gating_kernel,
        grid=grid,
        in_specs=[
            pl.BlockSpec((BLK, EMBED), lambda i: (i, 0)),
            pl.BlockSpec((EMBED, NUM_EXPERTS), lambda i: (0, 0)),
        ],
        out_specs=[
            pl.BlockSpec((BLK, TOPK), lambda i: (i, 0)),
            pl.BlockSpec((BLK, TOPK), lambda i: (i, 0)),
        ],
        out_shape=[
            jax.ShapeDtypeStruct((tokens, TOPK), jnp.float32),
            jax.ShapeDtypeStruct((tokens, TOPK), jnp.int32),
        ],
    )(query_flat, w_gate)
    return (gates, idx)


# speedup vs baseline: 1.0716x; 1.0018x over previous
import jax
import jax.numpy as jnp
from jax.experimental import pallas as pl
from jax.experimental.pallas import tpu as pltpu

EMBED = 4096
NUM_EXPERTS = 64
TOPK = 8
BLK = 1024


def _gating_kernel(q_ref, w_ref, gates_ref, idx_ref):
    q = q_ref[...]
    w = w_ref[...]
    lt = jax.lax.dot_general(w, q, (((0,), (1,)), ((), ())),
                             preferred_element_type=jnp.float32)
    m = jnp.max(lt, axis=0, keepdims=True)
    e = jnp.exp(lt - m)
    p = e / jnp.sum(e, axis=0, keepdims=True)

    iota = jax.lax.broadcasted_iota(jnp.int32, p.shape, 0)
    vals = p
    grows = []
    irows = []
    for _ in range(TOPK):
        mx = jnp.max(vals, axis=0, keepdims=True)
        amx = jnp.min(jnp.where(vals == mx, iota, NUM_EXPERTS), axis=0,
                      keepdims=True)
        grows.append(mx)
        irows.append(amx)
        vals = jnp.where(iota == amx, -jnp.inf, vals)
    gt = jnp.concatenate(grows, axis=0)
    it = jnp.concatenate(irows, axis=0)
    gates_ref[...] = gt.T
    idx_ref[...] = it.T


def kernel(query, w_gate):
    B, A, P, D = query.shape
    tokens = B * A * P
    query_flat = query.reshape(tokens, D)
    grid = (tokens // BLK,)
    gates, idx = pl.pallas_call(
        _gating_kernel,
        grid=grid,
        in_specs=[
            pl.BlockSpec((BLK, EMBED), lambda i: (i, 0)),
            pl.BlockSpec((EMBED, NUM_EXPERTS), lambda i: (0, 0)),
        ],
        out_specs=[
            pl.BlockSpec((BLK, TOPK), lambda i: (i, 0)),
            pl.BlockSpec((BLK, TOPK), lambda i: (i, 0)),
        ],
        out_shape=[
            jax.ShapeDtypeStruct((tokens, TOPK), jnp.float32),
            jax.ShapeDtypeStruct((tokens, TOPK), jnp.int32),
        ],
        compiler_params=pltpu.CompilerParams(
            dimension_semantics=("parallel",),
        ),
    )(query_flat, w_gate)
    return (gates, idx)
